# baseline (speedup 1.0000x reference)
import functools

import jax
import jax.numpy as jnp
from jax import lax
from jax.experimental import pallas as pl
from jax.experimental.pallas import tpu as pltpu
from jax.experimental.pallas import tpu_sc as plsc

_NUM_CORES = 2
_NUM_SUBCORES = 16
_NW = _NUM_CORES * _NUM_SUBCORES
_CHUNK = 128


def kernel(x, W):
    B, S = x.shape
    V, D = W.shape
    n = B * S
    per_w = n // _NW
    k = per_w // _CHUNK
    idx = x.reshape(_NW, k, _CHUNK)

    mesh = plsc.VectorSubcoreMesh(core_axis_name="c", subcore_axis_name="s")

    @functools.partial(
        pl.kernel,
        mesh=mesh,
        compiler_params=pltpu.CompilerParams(use_tc_tiling_on_sc=False),
        out_type=jax.ShapeDtypeStruct((n, D), W.dtype),
        scratch_types=[
            pltpu.VMEM((k, _CHUNK), jnp.int32),
            pltpu.VMEM((_CHUNK, D), jnp.float32),
            pltpu.SemaphoreType.DMA,
        ],
    )
    def gather_kernel(w_hbm, i_hbm, o_hbm, idx_v, rows_v, sem):
        wid = lax.axis_index("s") * _NUM_CORES + lax.axis_index("c")
        base = wid * per_w
        pltpu.sync_copy(i_hbm.at[wid], idx_v)

        @pl.loop(0, k)
        def _(j):
            pltpu.async_copy(w_hbm.at[idx_v.at[j]], rows_v, sem).wait()
            pltpu.sync_copy(rows_v, o_hbm.at[pl.ds(base + j * _CHUNK, _CHUNK)])

    out = gather_kernel(W, idx)
    return out.reshape(B, S, D)


# speedup vs baseline: 1.1165x; 1.1165x over previous
import functools

import jax
import jax.numpy as jnp
from jax import lax
from jax.experimental import pallas as pl
from jax.experimental.pallas import tpu as pltpu
from jax.experimental.pallas import tpu_sc as plsc

_NUM_CORES = 2
_NUM_SUBCORES = 16
_NW = _NUM_CORES * _NUM_SUBCORES
_CHUNK = 128
_G = 4


def kernel(x, W):
    B, S = x.shape
    V, D = W.shape
    n = B * S
    per_w = n // _NW
    k = per_w // _CHUNK
    nbatch = k // _G
    R = _G * _CHUNK
    idx = x.reshape(_NW, k, _CHUNK)

    mesh = plsc.VectorSubcoreMesh(core_axis_name="c", subcore_axis_name="s")

    @functools.partial(
        pl.kernel,
        mesh=mesh,
        compiler_params=pltpu.CompilerParams(use_tc_tiling_on_sc=False),
        out_type=jax.ShapeDtypeStruct((n, D), W.dtype),
        scratch_types=[
            pltpu.VMEM((k, _CHUNK), jnp.int32),
            pltpu.VMEM((2, R, D), jnp.float32),
            pltpu.SemaphoreType.DMA,
            pltpu.SemaphoreType.DMA,
            pltpu.SemaphoreType.DMA,
            pltpu.SemaphoreType.DMA,
        ],
    )
    def gather_kernel(w_hbm, i_hbm, o_hbm, idx_v, rows_v, sg0, sg1, sw0, sw1):
        wid = lax.axis_index("s") * _NUM_CORES + lax.axis_index("c")
        base = wid * per_w
        pltpu.sync_copy(i_hbm.at[wid], idx_v)

        def fire(t, g, sem):
            for u in range(_G):
                pltpu.async_copy(
                    w_hbm.at[idx_v.at[t * _G + u]],
                    rows_v.at[g].at[pl.ds(u * _CHUNK, _CHUNK)],
                    sem)

        def drain_gathers(g, sem):
            pltpu.make_async_copy(
                o_hbm.at[pl.ds(base, R)], rows_v.at[g], sem).wait()

        def write_start(t, g, sem):
            pltpu.async_copy(
                rows_v.at[g], o_hbm.at[pl.ds(base + t * R, R)], sem)

        def write_wait(g, sem):
            pltpu.make_async_copy(
                rows_v.at[g], o_hbm.at[pl.ds(base, R)], sem).wait()

        fire(0, 0, sg0)

        @pl.loop(0, nbatch, step=2)
        def _(t):
            @pl.when(t > 0)
            def _():
                write_wait(1, sw1)

            fire(t + 1, 1, sg1)
            drain_gathers(0, sg0)
            write_start(t, 0, sw0)

            @pl.when(t + 2 < nbatch)
            def _():
                write_wait(0, sw0)
                fire(t + 2, 0, sg0)

            drain_gathers(1, sg1)
            write_start(t + 1, 1, sw1)

        write_wait(0, sw0)
        write_wait(1, sw1)

    out = gather_kernel(W, idx)
    return out.reshape(B, S, D)


# speedup vs baseline: 1.1172x; 1.0007x over previous
import functools

import jax
import jax.numpy as jnp
from jax import lax
from jax.experimental import pallas as pl
from jax.experimental.pallas import tpu as pltpu
from jax.experimental.pallas import tpu_sc as plsc

_NUM_CORES = 2
_NUM_SUBCORES = 16
_NW = _NUM_CORES * _NUM_SUBCORES
_SPLITS = ((0, 128), (128, 72))
_G = 2


def kernel(x, W):
    B, S = x.shape
    V, D = W.shape
    rows_per_w = B // _NW
    ngroup = rows_per_w // _G

    mesh = plsc.VectorSubcoreMesh(core_axis_name="c", subcore_axis_name="s")

    @functools.partial(
        pl.kernel,
        mesh=mesh,
        compiler_params=pltpu.CompilerParams(use_tc_tiling_on_sc=False),
        out_type=jax.ShapeDtypeStruct((B, S, D), W.dtype),
        scratch_types=[
            pltpu.VMEM((rows_per_w, S), jnp.int32),
            pltpu.VMEM((2, _G, S, D), jnp.float32),
            pltpu.SemaphoreType.DMA,
            pltpu.SemaphoreType.DMA,
            pltpu.SemaphoreType.DMA,
            pltpu.SemaphoreType.DMA,
        ],
    )
    def gather_kernel(w_hbm, i_hbm, o_hbm, idx_v, rows_v, sg0, sg1, sw0, sw1):
        wid = lax.axis_index("s") * _NUM_CORES + lax.axis_index("c")
        base = wid * rows_per_w
        pltpu.sync_copy(i_hbm.at[pl.ds(base, rows_per_w)], idx_v)

        def fire(t, g, sem):
            for u in range(_G):
                for off, sz in _SPLITS:
                    pltpu.async_copy(
                        w_hbm.at[idx_v.at[t * _G + u].at[pl.ds(off, sz)]],
                        rows_v.at[g].at[u].at[pl.ds(off, sz)],
                        sem)

        def drain_gathers(g, sem):
            pltpu.make_async_copy(
                o_hbm.at[pl.ds(base, _G)], rows_v.at[g], sem).wait()

        def write_start(t, g, sem):
            pltpu.async_copy(
                rows_v.at[g], o_hbm.at[pl.ds(base + t * _G, _G)], sem)

        def write_wait(g, sem):
            pltpu.make_async_copy(
                rows_v.at[g], o_hbm.at[pl.ds(base, _G)], sem).wait()

        fire(0, 0, sg0)

        @pl.loop(0, ngroup, step=2)
        def _(t):
            @pl.when(t > 0)
            def _():
                write_wait(1, sw1)

            fire(t + 1, 1, sg1)
            drain_gathers(0, sg0)
            write_start(t, 0, sw0)

            @pl.when(t + 2 < ngroup)
            def _():
                write_wait(0, sw0)
                fire(t + 2, 0, sg0)

            drain_gathers(1, sg1)
            write_start(t + 1, 1, sw1)

        write_wait(0, sw0)
        write_wait(1, sw1)

    return gather_kernel(W, x)


# speedup vs baseline: 1.3634x; 1.2203x over previous
import functools

import jax
import jax.numpy as jnp
from jax import lax
from jax.experimental import pallas as pl
from jax.experimental.pallas import tpu as pltpu
from jax.experimental.pallas import tpu_sc as plsc

_NUM_CORES = 2
_NUM_SUBCORES = 16
_NW = _NUM_CORES * _NUM_SUBCORES
_SPLITS = ((0, 128), (128, 72))
_DP = 128


def kernel(x, W):
    B, S = x.shape
    V, D = W.shape
    rows_per_w = B // _NW
    W128 = jnp.pad(W, ((0, 0), (0, _DP - D)))

    mesh = plsc.VectorSubcoreMesh(core_axis_name="c", subcore_axis_name="s")

    @functools.partial(
        pl.kernel,
        mesh=mesh,
        compiler_params=pltpu.CompilerParams(use_tc_tiling_on_sc=False),
        out_type=jax.ShapeDtypeStruct((B, S, _DP), W.dtype),
        scratch_types=[
            pltpu.VMEM((rows_per_w, S), jnp.int32),
            pltpu.VMEM((2, S, _DP), jnp.float32),
            pltpu.SemaphoreType.DMA,
            pltpu.SemaphoreType.DMA,
            pltpu.SemaphoreType.DMA,
            pltpu.SemaphoreType.DMA,
        ],
    )
    def gather_kernel(w_hbm, i_hbm, o_hbm, idx_v, rows_v, sg0, sg1, sw0, sw1):
        wid = lax.axis_index("s") * _NUM_CORES + lax.axis_index("c")
        base = wid * rows_per_w
        pltpu.sync_copy(i_hbm.at[pl.ds(base, rows_per_w)], idx_v)

        def fire(t, g, sem):
            for off, sz in _SPLITS:
                pltpu.async_copy(
                    w_hbm.at[idx_v.at[t].at[pl.ds(off, sz)]],
                    rows_v.at[g].at[pl.ds(off, sz)],
                    sem)

        def drain_gathers(g, sem):
            pltpu.make_async_copy(
                o_hbm.at[base], rows_v.at[g], sem).wait()

        def write_start(t, g, sem):
            pltpu.async_copy(rows_v.at[g], o_hbm.at[base + t], sem)

        def write_wait(g, sem):
            pltpu.make_async_copy(
                rows_v.at[g], o_hbm.at[base], sem).wait()

        fire(0, 0, sg0)

        @pl.loop(0, rows_per_w, step=2)
        def _(t):
            @pl.when(t > 0)
            def _():
                write_wait(1, sw1)

            fire(t + 1, 1, sg1)
            drain_gathers(0, sg0)
            write_start(t, 0, sw0)

            @pl.when(t + 2 < rows_per_w)
            def _():
                write_wait(0, sw0)
                fire(t + 2, 0, sg0)

            drain_gathers(1, sg1)
            write_start(t + 1, 1, sw1)

        write_wait(0, sw0)
        write_wait(1, sw1)

    out = gather_kernel(W128, x)
    return out[:, :, :D]
